# baseline (speedup 1.0000x reference)
import functools

import jax
import jax.numpy as jnp
from jax import lax
from jax.experimental import pallas as pl
from jax.experimental.pallas import tpu as pltpu
from jax.experimental.pallas import tpu_sc as plsc

DIM = 32
LANES = 16
CHUNK = 128


def kernel(h, r, t, E, R):
    B = h.shape[0]
    mesh = plsc.VectorSubcoreMesh(core_axis_name="c", subcore_axis_name="s")
    NW = mesh.num_cores * mesh.num_subcores
    b_per_w = B // NW
    n_chunks = b_per_w // CHUNK

    h3 = h.reshape(NW, n_chunks, CHUNK)
    r3 = r.reshape(NW, n_chunks, CHUNK)
    t3 = t.reshape(NW, n_chunks, CHUNK)

    @functools.partial(
        pl.kernel,
        out_type=jax.ShapeDtypeStruct((B,), jnp.float32),
        mesh=mesh,
        scratch_types=[
            pltpu.VMEM((n_chunks, CHUNK), jnp.int32),
            pltpu.VMEM((n_chunks, CHUNK), jnp.int32),
            pltpu.VMEM((n_chunks, CHUNK), jnp.int32),
            pltpu.VMEM((n_chunks, CHUNK, DIM), jnp.float32),
            pltpu.VMEM((n_chunks, CHUNK, DIM), jnp.float32),
            pltpu.VMEM((n_chunks, CHUNK, DIM), jnp.float32),
            pltpu.VMEM((b_per_w,), jnp.float32),
            pltpu.SemaphoreType.DMA,
        ],
        compiler_params=pltpu.CompilerParams(
            needs_layout_passes=False, use_tc_tiling_on_sc=False),
    )
    def transe(h_hbm, r_hbm, t_hbm, E_hbm, R_hbm, out_hbm,
               h_v, r_v, t_v, eh_v, rr_v, et_v, out_v, sem):
        wid = lax.axis_index("s") * mesh.num_cores + lax.axis_index("c")
        base = wid * b_per_w

        pltpu.sync_copy(h_hbm.at[wid], h_v)
        pltpu.sync_copy(r_hbm.at[wid], r_v)
        pltpu.sync_copy(t_hbm.at[wid], t_v)

        copies = []
        for j in range(n_chunks):
            copies.append(pltpu.async_copy(E_hbm.at[h_v.at[j]], eh_v.at[j], sem))
            copies.append(pltpu.async_copy(R_hbm.at[r_v.at[j]], rr_v.at[j], sem))
            copies.append(pltpu.async_copy(E_hbm.at[t_v.at[j]], et_v.at[j], sem))
        for c in copies:
            c.wait()

        lane = lax.iota(jnp.int32, LANES)
        for j in range(n_chunks):
            def group_body(g, _, j=j):
                acc = jnp.zeros((LANES,), jnp.float32)
                for k in range(LANES):
                    i = g * LANES + k
                    a0 = eh_v[j, i, pl.ds(0, LANES)]
                    a1 = eh_v[j, i, pl.ds(LANES, LANES)]
                    b0 = rr_v[j, i, pl.ds(0, LANES)]
                    b1 = rr_v[j, i, pl.ds(LANES, LANES)]
                    c0 = et_v[j, i, pl.ds(0, LANES)]
                    c1 = et_v[j, i, pl.ds(LANES, LANES)]
                    s = jnp.abs(a0 + b0 - c0) + jnp.abs(a1 + b1 - c1)
                    acc = jnp.where(lane == k, jnp.sum(s), acc)
                out_v[pl.ds(j * CHUNK + g * LANES, LANES)] = acc
                return _
            lax.fori_loop(0, CHUNK // LANES, group_body, None)

        pltpu.sync_copy(out_v, out_hbm.at[pl.ds(base, b_per_w)])

    return transe(h3, r3, t3, E, R)
